# baseline (speedup 1.0000x reference)
import functools

import jax
import jax.numpy as jnp
from jax import lax
from jax.experimental import pallas as pl
from jax.experimental.pallas import tpu as pltpu
from jax.experimental.pallas import tpu_sc as plsc

_K = 1024
_D = 256
_R = 512
_N = 16384

_NC = 2
_NS = 16
_NW = _NC * _NS
_BW = _N // _NW
_CH = 128
_NCH = _BW // _CH


def _vq_block(x_ref, et_ref, e2_ref,
              enc_ref, idx_ref, loss_ref):
    i = pl.program_id(0)
    x = x_ref[...]
    x2 = jnp.sum(x ** 2, axis=1, keepdims=True)
    minv = x2
    enc_ref[...] = jnp.zeros((_R, _K), jnp.float32)
    idx_ref[0, 0, :] = x2[:, 0].astype(jnp.int32)

    @pl.when(i == 0)
    def _():
        loss_ref[...] = jnp.zeros((1, 1), jnp.float32)

    loss_ref[...] += jnp.sum(minv).reshape(1, 1)


def _sc_gather(emb_hbm, idx_hbm, out_hbm, idx_v, rows_v, sem):
    wid = lax.axis_index("s") * _NC + lax.axis_index("c")
    base = wid * _BW
    for c in range(_NCH):
        off = base + c * _CH
        pltpu.sync_copy(idx_hbm.at[pl.ds(off, _CH)], idx_v)
        pltpu.async_copy(emb_hbm.at[idx_v], rows_v, sem).wait()
        pltpu.sync_copy(rows_v, out_hbm.at[pl.ds(off, _CH)])


def kernel(inputs, embedding):
    input_shape = inputs.shape
    flat = inputs.reshape(-1, _D)
    e2 = jnp.sum(embedding ** 2, axis=1)[None, :]
    et = embedding.T

    grid = _N // _R
    enc, idx3, loss_acc = pl.pallas_call(
        _vq_block,
        grid=(grid,),
        in_specs=[
            pl.BlockSpec((_R, _D), lambda i: (i, 0)),
            pl.BlockSpec((_D, _K), lambda i: (0, 0)),
            pl.BlockSpec((1, _K), lambda i: (0, 0)),
        ],
        out_specs=[
            pl.BlockSpec((_R, _K), lambda i: (i, 0)),
            pl.BlockSpec((1, 1, _R), lambda i: (i, 0, 0)),
            pl.BlockSpec((1, 1), lambda i: (0, 0)),
        ],
        out_shape=[
            jax.ShapeDtypeStruct((_N, _K), jnp.float32),
            jax.ShapeDtypeStruct((grid, 1, _R), jnp.int32),
            jax.ShapeDtypeStruct((1, 1), jnp.float32),
        ],
        compiler_params=pltpu.CompilerParams(
            dimension_semantics=("arbitrary",),
        ),
    )(flat, et, e2)

    idx_flat = idx3.reshape(_N)
    gather = functools.partial(
        pl.kernel,
        mesh=plsc.VectorSubcoreMesh(core_axis_name="c", subcore_axis_name="s"),
        out_type=jax.ShapeDtypeStruct((_N, _D), jnp.float32),
        scratch_types=[
            pltpu.VMEM((_CH,), jnp.int32),
            pltpu.VMEM((_CH, _D), jnp.float32),
            pltpu.SemaphoreType.DMA,
        ],
    )(_sc_gather)
    q = gather(embedding, idx_flat)

    mean_sq = loss_acc[0, 0] / (_N * _D)
    loss = mean_sq + 0.25 * mean_sq
    quantized = q.reshape(input_shape)
    encoding_indices = idx3.reshape(input_shape[:-1])
    return (quantized, loss, enc, encoding_indices)


# speedup vs baseline: 1.1926x; 1.1926x over previous
import functools

import jax
import jax.numpy as jnp
from jax import lax
from jax.experimental import pallas as pl
from jax.experimental.pallas import tpu as pltpu
from jax.experimental.pallas import tpu_sc as plsc

_K = 1024
_D = 256
_R = 1024
_N = 16384

_NC = 2
_NS = 16
_NW = _NC * _NS
_BW = _N // _NW
_CH = 128
_NCH = _BW // _CH


def _vq_block(x_ref, et_ref, e2_ref,
              enc_ref, idx_ref, loss_ref):
    i = pl.program_id(0)
    x = x_ref[...]
    x2 = jnp.sum(x ** 2, axis=1, keepdims=True)
    m = jnp.dot(x, et_ref[...],
                preferred_element_type=jnp.float32)
    d = (x2 + e2_ref[...]) - 2.0 * m
    minv = jnp.min(d, axis=1, keepdims=True)
    iota_f = lax.broadcasted_iota(
        jnp.int32, (_R, _K), 1).astype(jnp.float32)
    idx_f = jnp.min(jnp.where(d == minv, iota_f, float(_K)), axis=1)
    enc_ref[...] = (iota_f == idx_f[:, None]).astype(jnp.float32)
    idx_ref[0, 0, :] = idx_f.astype(jnp.int32)

    @pl.when(i == 0)
    def _():
        loss_ref[...] = jnp.zeros((1, 1), jnp.float32)

    loss_ref[...] += jnp.sum(minv).reshape(1, 1)


def _sc_gather(emb_hbm, idx_hbm, out_hbm, idx_v, rows_v, sem):
    wid = lax.axis_index("s") * _NC + lax.axis_index("c")
    base = wid * _BW
    for c in range(_NCH):
        off = base + c * _CH
        pltpu.sync_copy(idx_hbm.at[pl.ds(off, _CH)], idx_v)
        pltpu.async_copy(emb_hbm.at[idx_v], rows_v, sem).wait()
        pltpu.sync_copy(rows_v, out_hbm.at[pl.ds(off, _CH)])


def kernel(inputs, embedding):
    input_shape = inputs.shape
    flat = inputs.reshape(-1, _D)
    e2 = jnp.sum(embedding ** 2, axis=1)[None, :]
    et = embedding.T

    grid = _N // _R
    enc, idx3, loss_acc = pl.pallas_call(
        _vq_block,
        grid=(grid,),
        in_specs=[
            pl.BlockSpec((_R, _D), lambda i: (i, 0)),
            pl.BlockSpec((_D, _K), lambda i: (0, 0)),
            pl.BlockSpec((1, _K), lambda i: (0, 0)),
        ],
        out_specs=[
            pl.BlockSpec((_R, _K), lambda i: (i, 0)),
            pl.BlockSpec((1, 1, _R), lambda i: (i, 0, 0)),
            pl.BlockSpec((1, 1), lambda i: (0, 0)),
        ],
        out_shape=[
            jax.ShapeDtypeStruct((_N, _K), jnp.float32),
            jax.ShapeDtypeStruct((grid, 1, _R), jnp.int32),
            jax.ShapeDtypeStruct((1, 1), jnp.float32),
        ],
        compiler_params=pltpu.CompilerParams(
            dimension_semantics=("arbitrary",),
        ),
    )(flat, et, e2)

    idx_flat = idx3.reshape(_N)
    gather = functools.partial(
        pl.kernel,
        mesh=plsc.VectorSubcoreMesh(core_axis_name="c", subcore_axis_name="s"),
        out_type=jax.ShapeDtypeStruct((_N, _D), jnp.float32),
        scratch_types=[
            pltpu.VMEM((_CH,), jnp.int32),
            pltpu.VMEM((_CH, _D), jnp.float32),
            pltpu.SemaphoreType.DMA,
        ],
    )(_sc_gather)
    q = gather(embedding, idx_flat)

    mean_sq = loss_acc[0, 0] / (_N * _D)
    loss = mean_sq + 0.25 * mean_sq
    quantized = q.reshape(input_shape)
    encoding_indices = idx3.reshape(input_shape[:-1])
    return (quantized, loss, enc, encoding_indices)


# speedup vs baseline: 1.2142x; 1.0181x over previous
import functools

import jax
import jax.numpy as jnp
from jax import lax
from jax.experimental import pallas as pl
from jax.experimental.pallas import tpu as pltpu
from jax.experimental.pallas import tpu_sc as plsc

_K = 1024
_D = 256
_R = 2048
_N = 16384

_NC = 2
_NS = 16
_NW = _NC * _NS
_BW = _N // _NW
_CH = 128
_NCH = _BW // _CH


def _vq_block(x_ref, et_ref, e2_ref,
              enc_ref, idx_ref, loss_ref):
    i = pl.program_id(0)
    x = x_ref[...]
    x2 = jnp.sum(x ** 2, axis=1, keepdims=True)
    m = jnp.dot(x, et_ref[...],
                preferred_element_type=jnp.float32)
    d = (x2 + e2_ref[...]) - 2.0 * m
    minv = jnp.min(d, axis=1, keepdims=True)
    iota_f = lax.broadcasted_iota(
        jnp.int32, (_R, _K), 1).astype(jnp.float32)
    idx_f = jnp.min(jnp.where(d == minv, iota_f, float(_K)), axis=1)
    enc_ref[...] = (iota_f == idx_f[:, None]).astype(jnp.float32)
    idx_ref[0, 0, :] = idx_f.astype(jnp.int32)

    @pl.when(i == 0)
    def _():
        loss_ref[...] = jnp.zeros((1, 1), jnp.float32)

    loss_ref[...] += jnp.sum(minv).reshape(1, 1)


def _sc_gather(emb_hbm, idx_hbm, out_hbm, idx_v, rows_v, sem):
    wid = lax.axis_index("s") * _NC + lax.axis_index("c")
    base = wid * _BW
    for c in range(_NCH):
        off = base + c * _CH
        pltpu.sync_copy(idx_hbm.at[pl.ds(off, _CH)], idx_v)
        pltpu.async_copy(emb_hbm.at[idx_v], rows_v, sem).wait()
        pltpu.sync_copy(rows_v, out_hbm.at[pl.ds(off, _CH)])


def kernel(inputs, embedding):
    input_shape = inputs.shape
    flat = inputs.reshape(-1, _D)
    e2 = jnp.sum(embedding ** 2, axis=1)[None, :]
    et = embedding.T

    grid = _N // _R
    enc, idx3, loss_acc = pl.pallas_call(
        _vq_block,
        grid=(grid,),
        in_specs=[
            pl.BlockSpec((_R, _D), lambda i: (i, 0)),
            pl.BlockSpec((_D, _K), lambda i: (0, 0)),
            pl.BlockSpec((1, _K), lambda i: (0, 0)),
        ],
        out_specs=[
            pl.BlockSpec((_R, _K), lambda i: (i, 0)),
            pl.BlockSpec((1, 1, _R), lambda i: (i, 0, 0)),
            pl.BlockSpec((1, 1), lambda i: (0, 0)),
        ],
        out_shape=[
            jax.ShapeDtypeStruct((_N, _K), jnp.float32),
            jax.ShapeDtypeStruct((grid, 1, _R), jnp.int32),
            jax.ShapeDtypeStruct((1, 1), jnp.float32),
        ],
        compiler_params=pltpu.CompilerParams(
            dimension_semantics=("arbitrary",),
        ),
    )(flat, et, e2)

    idx_flat = idx3.reshape(_N)
    gather = functools.partial(
        pl.kernel,
        mesh=plsc.VectorSubcoreMesh(core_axis_name="c", subcore_axis_name="s"),
        out_type=jax.ShapeDtypeStruct((_N, _D), jnp.float32),
        scratch_types=[
            pltpu.VMEM((_CH,), jnp.int32),
            pltpu.VMEM((_CH, _D), jnp.float32),
            pltpu.SemaphoreType.DMA,
        ],
    )(_sc_gather)
    q = gather(embedding, idx_flat)

    mean_sq = loss_acc[0, 0] / (_N * _D)
    loss = mean_sq + 0.25 * mean_sq
    quantized = q.reshape(input_shape)
    encoding_indices = idx3.reshape(input_shape[:-1])
    return (quantized, loss, enc, encoding_indices)


# speedup vs baseline: 1.3440x; 1.1069x over previous
import functools

import jax
import jax.numpy as jnp
from jax import lax
from jax.experimental import pallas as pl
from jax.experimental.pallas import tpu as pltpu
from jax.experimental.pallas import tpu_sc as plsc

_K = 1024
_D = 256
_R = 2048
_N = 16384

_NC = 2
_NS = 16
_NW = _NC * _NS
_BW = _N // _NW
_CH = 128
_NCH = _BW // _CH


def _vq_block(x_ref, et_ref, e2_ref,
              enc_ref, idx_ref, loss_ref):
    i = pl.program_id(0)
    x = x_ref[...]
    x2 = jnp.sum(x ** 2, axis=1, keepdims=True)
    m = jnp.dot(x, et_ref[...],
                preferred_element_type=jnp.float32)
    d = (x2 + e2_ref[...]) - 2.0 * m
    minv = jnp.min(d, axis=1, keepdims=True)
    iota_f = lax.broadcasted_iota(
        jnp.int32, (_R, _K), 1).astype(jnp.float32)
    idx_f = jnp.min(jnp.where(d == minv, iota_f, float(_K)), axis=1)
    enc_ref[...] = (iota_f == idx_f[:, None]).astype(jnp.float32)
    idx_ref[...] = idx_f.astype(jnp.int32)[:, None]

    @pl.when(i == 0)
    def _():
        loss_ref[...] = jnp.zeros((1, 1), jnp.float32)

    loss_ref[...] += jnp.sum(minv).reshape(1, 1)


def _sc_gather(emb_hbm, idx_hbm, out_hbm, idx_v, rows_v, sem):
    wid = lax.axis_index("s") * _NC + lax.axis_index("c")
    base = wid * _BW
    for c in range(_NCH):
        off = base + c * _CH
        pltpu.sync_copy(idx_hbm.at[pl.ds(off, _CH)], idx_v)
        pltpu.async_copy(emb_hbm.at[idx_v], rows_v, sem).wait()
        pltpu.sync_copy(rows_v, out_hbm.at[pl.ds(off, _CH)])


def kernel(inputs, embedding):
    input_shape = inputs.shape
    flat = inputs.reshape(-1, _D)
    e2 = jnp.sum(embedding ** 2, axis=1)[None, :]
    et = embedding.T

    grid = _N // _R
    enc, idx3, loss_acc = pl.pallas_call(
        _vq_block,
        grid=(grid,),
        in_specs=[
            pl.BlockSpec((_R, _D), lambda i: (i, 0)),
            pl.BlockSpec((_D, _K), lambda i: (0, 0)),
            pl.BlockSpec((1, _K), lambda i: (0, 0)),
        ],
        out_specs=[
            pl.BlockSpec((_R, _K), lambda i: (i, 0)),
            pl.BlockSpec((_R, 1), lambda i: (i, 0)),
            pl.BlockSpec((1, 1), lambda i: (0, 0)),
        ],
        out_shape=[
            jax.ShapeDtypeStruct((_N, _K), jnp.float32),
            jax.ShapeDtypeStruct((_N, 1), jnp.int32),
            jax.ShapeDtypeStruct((1, 1), jnp.float32),
        ],
        compiler_params=pltpu.CompilerParams(
            dimension_semantics=("arbitrary",),
        ),
    )(flat, et, e2)

    idx_flat = idx3.reshape(_N)
    gather = functools.partial(
        pl.kernel,
        mesh=plsc.VectorSubcoreMesh(core_axis_name="c", subcore_axis_name="s"),
        out_type=jax.ShapeDtypeStruct((_N, _D), jnp.float32),
        scratch_types=[
            pltpu.VMEM((_CH,), jnp.int32),
            pltpu.VMEM((_CH, _D), jnp.float32),
            pltpu.SemaphoreType.DMA,
        ],
    )(_sc_gather)
    q = gather(embedding, idx_flat)

    mean_sq = loss_acc[0, 0] / (_N * _D)
    loss = mean_sq + 0.25 * mean_sq
    quantized = q.reshape(input_shape)
    encoding_indices = idx3.reshape(input_shape[:-1])
    return (quantized, loss, enc, encoding_indices)
